# Initial kernel scaffold; baseline (speedup 1.0000x reference)
import functools

import jax
import jax.numpy as jnp
from jax import lax
from jax.experimental import pallas as pl
from jax.experimental.pallas import tpu as pltpu
from jax.experimental.pallas import tpu_sc as plsc

N = 10000
E = 160000
D = 256
DH = 128
NC = 2
NS = 16
L = 16
CHUNK = 128
CPT = 79
E_PAD = NS * CPT * CHUNK
N_PAD = 10240
NBLK = N_PAD // CHUNK
BPT = NBLK // NS
RPT = N_PAD // NS

_mesh = plsc.VectorSubcoreMesh(core_axis_name="c", subcore_axis_name="s")


@functools.partial(
    pl.kernel,
    out_type=jax.ShapeDtypeStruct((2, N_PAD, 1), jnp.float32),
    mesh=_mesh,
    scratch_types=[
        pltpu.VMEM((CPT, CHUNK), jnp.int32),
        pltpu.VMEM((CHUNK, 1), jnp.float32),
        pltpu.VMEM((RPT, 1), jnp.float32),
        pltpu.VMEM_SHARED((N_PAD, 1), jnp.float32),
    ],
)
def _degrees(eidx_hbm, deg_hbm, idx_v, ones_v, row_v, hist_sh):
    c = lax.axis_index("c")
    s = lax.axis_index("s")

    @pl.loop(0, RPT, step=L)
    def _(i):
        row_v[pl.ds(i, L), 0] = jnp.zeros((L,), jnp.float32)

    @pl.loop(0, CHUNK, step=L)
    def _(i):
        ones_v[pl.ds(i, L), 0] = jnp.ones((L,), jnp.float32)

    pltpu.sync_copy(row_v, hist_sh.at[pl.ds(s * RPT, RPT)])
    pltpu.sync_copy(eidx_hbm.at[c].at[s], idx_v)
    plsc.subcore_barrier()

    @pl.loop(0, CPT)
    def _(j):
        pltpu.sync_copy(ones_v, hist_sh.at[idx_v.at[j]], add=True)

    plsc.subcore_barrier()
    pltpu.sync_copy(hist_sh.at[pl.ds(s * RPT, RPT)], row_v)
    pltpu.sync_copy(row_v, deg_hbm.at[c].at[pl.ds(s * RPT, RPT)])


_RB = 1280


def _matmul_body(x_ref, dsrc_ref, ddst_ref, w_ref, y_ref, nd_ref):
    ds_ = dsrc_ref[...]
    ns_ = jnp.where(ds_ > 0, lax.rsqrt(jnp.maximum(ds_, 1e-12)), 0.0)
    dd_ = ddst_ref[...]
    nd_ref[...] = jnp.where(dd_ > 0, lax.rsqrt(jnp.maximum(dd_, 1e-12)), 0.0)
    h = x_ref[...] * ns_
    y = jnp.dot(h, w_ref[...], preferred_element_type=jnp.float32)
    y_ref[0] = y[:, :DH]
    y_ref[1] = y[:, DH:]


_matmul = pl.pallas_call(
    _matmul_body,
    grid=(N_PAD // _RB,),
    in_specs=[
        pl.BlockSpec((_RB, D), lambda i: (i, 0)),
        pl.BlockSpec((_RB, 1), lambda i: (i, 0)),
        pl.BlockSpec((_RB, 1), lambda i: (i, 0)),
        pl.BlockSpec((D, D), lambda i: (0, 0)),
    ],
    out_specs=[
        pl.BlockSpec((2, _RB, DH), lambda i: (0, i, 0)),
        pl.BlockSpec((_RB, 1), lambda i: (i, 0)),
    ],
    out_shape=[
        jax.ShapeDtypeStruct((2, N_PAD, DH), jnp.float32),
        jax.ShapeDtypeStruct((N_PAD, 1), jnp.float32),
    ],
)


@functools.partial(
    pl.kernel,
    out_type=jax.ShapeDtypeStruct((N_PAD, D), jnp.float32),
    mesh=_mesh,
    scratch_types=[
        pltpu.VMEM((CPT, CHUNK), jnp.int32),
        pltpu.VMEM((CPT, CHUNK), jnp.int32),
        pltpu.VMEM((CHUNK, DH), jnp.float32),
        pltpu.VMEM((CHUNK, DH), jnp.float32),
        pltpu.VMEM((DH,), jnp.float32),
        pltpu.SMEM((CHUNK,), jnp.float32),
        pltpu.VMEM_SHARED((N_PAD, DH), jnp.float32),
    ],
)
def _edge_pass(y_hbm, eidx_hbm, nrm_hbm, b_hbm, out_hbm,
               src_v, dst_v, gbuf, rbuf, b_v, nrm_s, acc_sh):
    c = lax.axis_index("c")
    s = lax.axis_index("s")

    @pl.loop(0, CHUNK)
    def _(r):
        @pl.loop(0, DH, step=L)
        def _(q):
            rbuf[r, pl.ds(q, L)] = jnp.zeros((L,), jnp.float32)

    @pl.loop(0, RPT, step=CHUNK)
    def _(r0):
        pltpu.sync_copy(rbuf, acc_sh.at[pl.ds(s * RPT + r0, CHUNK), :])

    pltpu.sync_copy(eidx_hbm.at[0].at[s], src_v)
    pltpu.sync_copy(eidx_hbm.at[1].at[s], dst_v)
    pltpu.sync_copy(b_hbm.at[c], b_v)
    plsc.subcore_barrier()

    @pl.loop(0, CPT)
    def _(j):
        pltpu.sync_copy(y_hbm.at[c].at[src_v.at[j]], gbuf)
        pltpu.sync_copy(gbuf, acc_sh.at[dst_v.at[j]], add=True)

    plsc.subcore_barrier()

    @pl.loop(0, BPT)
    def _(kb):
        blk = s * BPT + kb
        r0 = blk * CHUNK
        pltpu.sync_copy(acc_sh.at[pl.ds(r0, CHUNK), :], rbuf)
        pltpu.sync_copy(nrm_hbm.at[blk], nrm_s)

        @pl.loop(0, CHUNK)
        def _(r):
            scale = nrm_s[r]

            @pl.loop(0, DH, step=L)
            def _(q):
                rbuf[r, pl.ds(q, L)] = rbuf[r, pl.ds(q, L)] * scale + b_v[pl.ds(q, L)]

        pltpu.sync_copy(rbuf, out_hbm.at[pl.ds(r0, CHUNK), pl.ds(c * DH, DH)])


def kernel(x, edge_index, W, b):
    pad = jnp.full((E_PAD - E,), N, jnp.int32)
    eidx = jnp.stack([
        jnp.concatenate([edge_index[0], pad]).reshape(NS, CPT, CHUNK),
        jnp.concatenate([edge_index[1], pad]).reshape(NS, CPT, CHUNK),
    ])
    x_p = jnp.pad(x, ((0, N_PAD - N), (0, 0)))
    deg = _degrees(eidx)
    y, norm_dst = _matmul(x_p, deg[0], deg[1], W)
    out_pad = _edge_pass(y, eidx, norm_dst.reshape(NBLK, CHUNK),
                         b.reshape(NC, DH))
    return out_pad[:N]


# speedup vs baseline: 4.7857x; 4.7857x over previous
import functools

import jax
import jax.numpy as jnp
from jax import lax
from jax.experimental import pallas as pl
from jax.experimental.pallas import tpu as pltpu
from jax.experimental.pallas import tpu_sc as plsc

N = 10000
E = 160000
D = 256
DH = 128
NC = 2
NS = 16
L = 16
CHUNK = 128
CPT = 79
E_PAD = NS * CPT * CHUNK
N_PAD = 10240
NBLK = N_PAD // CHUNK
BPT = NBLK // NS
RPT = N_PAD // NS

_mesh = plsc.VectorSubcoreMesh(core_axis_name="c", subcore_axis_name="s")


@functools.partial(
    pl.kernel,
    out_type=jax.ShapeDtypeStruct((2, N_PAD), jnp.float32),
    mesh=_mesh,
    scratch_types=[
        pltpu.VMEM((CPT, CHUNK), jnp.int32),
        pltpu.VMEM((CHUNK,), jnp.float32),
        pltpu.VMEM((RPT,), jnp.float32),
        pltpu.VMEM_SHARED((N_PAD,), jnp.float32),
    ],
)
def _degrees(eidx_hbm, deg_hbm, idx_v, ones_v, row_v, hist_sh):
    c = lax.axis_index("c")
    s = lax.axis_index("s")

    @pl.loop(0, RPT, step=L)
    def _(i):
        row_v[pl.ds(i, L)] = jnp.zeros((L,), jnp.float32)

    @pl.loop(0, CHUNK, step=L)
    def _(i):
        ones_v[pl.ds(i, L)] = jnp.ones((L,), jnp.float32)

    pltpu.sync_copy(row_v, hist_sh.at[pl.ds(s * RPT, RPT)])
    pltpu.sync_copy(eidx_hbm.at[c].at[s], idx_v)
    plsc.subcore_barrier()

    @pl.loop(0, CPT)
    def _(j):
        pltpu.sync_copy(ones_v, hist_sh.at[idx_v.at[j]], add=True)

    plsc.subcore_barrier()
    pltpu.sync_copy(hist_sh.at[pl.ds(s * RPT, RPT)], row_v)
    pltpu.sync_copy(row_v, deg_hbm.at[c].at[pl.ds(s * RPT, RPT)])


_RB = 1280


def _matmul_body(x_ref, dsrc_ref, ddst_ref, w_ref, y_ref, nd_ref):
    ds_ = dsrc_ref[...]
    ns_ = jnp.where(ds_ > 0, lax.rsqrt(jnp.maximum(ds_, 1e-12)), 0.0)
    dd_ = ddst_ref[...]
    nd_ref[...] = jnp.where(dd_ > 0, lax.rsqrt(jnp.maximum(dd_, 1e-12)), 0.0)
    h = x_ref[...] * ns_
    y = jnp.dot(h, w_ref[...], preferred_element_type=jnp.float32)
    y_ref[0] = y[:, :DH]
    y_ref[1] = y[:, DH:]


_matmul = pl.pallas_call(
    _matmul_body,
    grid=(N_PAD // _RB,),
    in_specs=[
        pl.BlockSpec((_RB, D), lambda i: (i, 0)),
        pl.BlockSpec((_RB, 1), lambda i: (i, 0)),
        pl.BlockSpec((_RB, 1), lambda i: (i, 0)),
        pl.BlockSpec((D, D), lambda i: (0, 0)),
    ],
    out_specs=[
        pl.BlockSpec((2, _RB, DH), lambda i: (0, i, 0)),
        pl.BlockSpec((_RB, 1), lambda i: (i, 0)),
    ],
    out_shape=[
        jax.ShapeDtypeStruct((2, N_PAD, DH), jnp.float32),
        jax.ShapeDtypeStruct((N_PAD, 1), jnp.float32),
    ],
)


@functools.partial(
    pl.kernel,
    out_type=jax.ShapeDtypeStruct((N_PAD, D), jnp.float32),
    mesh=_mesh,
    scratch_types=[
        pltpu.VMEM((CPT, CHUNK), jnp.int32),
        pltpu.VMEM((CPT, CHUNK), jnp.int32),
        pltpu.VMEM((CHUNK, DH), jnp.float32),
        pltpu.VMEM((DH,), jnp.float32),
        pltpu.VMEM((CHUNK,), jnp.float32),
        pltpu.VMEM_SHARED((N_PAD, DH), jnp.float32),
    ],
)
def _edge_pass(y_hbm, eidx_hbm, nrm_hbm, b_hbm, out_hbm,
               src_v, dst_v, rbuf, b_v, nrm_v, acc_sh):
    c = lax.axis_index("c")
    s = lax.axis_index("s")

    @pl.loop(0, CHUNK)
    def _(r):
        @pl.loop(0, DH, step=L)
        def _(q):
            rbuf[r, pl.ds(q, L)] = jnp.zeros((L,), jnp.float32)

    @pl.loop(0, RPT, step=CHUNK)
    def _(r0):
        pltpu.sync_copy(rbuf, acc_sh.at[pl.ds(s * RPT + r0, CHUNK), :])

    pltpu.sync_copy(eidx_hbm.at[0].at[s], src_v)
    pltpu.sync_copy(eidx_hbm.at[1].at[s], dst_v)
    pltpu.sync_copy(b_hbm.at[c], b_v)
    plsc.subcore_barrier()

    @pl.loop(0, CPT)
    def _(j):
        pltpu.sync_copy(y_hbm.at[c].at[src_v.at[j]], rbuf)
        pltpu.sync_copy(rbuf, acc_sh.at[dst_v.at[j]], add=True)

    plsc.subcore_barrier()

    @pl.loop(0, BPT)
    def _(kb):
        blk = s * BPT + kb
        r0 = blk * CHUNK
        pltpu.sync_copy(acc_sh.at[pl.ds(r0, CHUNK), :], rbuf)
        pltpu.sync_copy(nrm_hbm.at[blk], nrm_v)

        @pl.loop(0, CHUNK, step=L)
        def _(r):
            nvec = nrm_v[pl.ds(r, L)]
            for k in range(L):
                scale = nvec[k]

                @pl.loop(0, DH, step=L)
                def _(q):
                    rbuf[r + k, pl.ds(q, L)] = (
                        rbuf[r + k, pl.ds(q, L)] * scale + b_v[pl.ds(q, L)])

        pltpu.sync_copy(rbuf, out_hbm.at[pl.ds(r0, CHUNK), pl.ds(c * DH, DH)])


def kernel(x, edge_index, W, b):
    pad = jnp.full((E_PAD - E,), N, jnp.int32)
    eidx = jnp.stack([
        jnp.concatenate([edge_index[0], pad]).reshape(NS, CPT, CHUNK),
        jnp.concatenate([edge_index[1], pad]).reshape(NS, CPT, CHUNK),
    ])
    x_p = jnp.pad(x, ((0, N_PAD - N), (0, 0)))
    deg = _degrees(eidx)
    y, norm_dst = _matmul(x_p, deg[0].reshape(N_PAD, 1),
                          deg[1].reshape(N_PAD, 1), W)
    out_pad = _edge_pass(y, eidx, norm_dst.reshape(NBLK, CHUNK),
                         b.reshape(NC, DH))
    return out_pad[:N]
